# baseline (speedup 1.0000x reference)
import functools

import jax
import jax.numpy as jnp
from jax import lax
from jax.experimental import pallas as pl
from jax.experimental.pallas import tpu as pltpu
from jax.experimental.pallas import tpu_sc as plsc

_NC = 2
_NS = 16
_NW = _NC * _NS

_EMBED = 128
_GROUP = 160


def _proj_body(table_ref, w_ref, b_ref, out_ref):
    out_ref[...] = (
        jnp.dot(table_ref[...], w_ref[...], preferred_element_type=jnp.float32)
        + b_ref[...]
    )


def _project_table(table, W, b):
    V, D = table.shape
    E = W.shape[1]
    blk = 2200
    assert V % blk == 0
    return pl.pallas_call(
        _proj_body,
        grid=(V // blk,),
        in_specs=[
            pl.BlockSpec((blk, D), lambda i: (i, 0)),
            pl.BlockSpec((D, E), lambda i: (0, 0)),
            pl.BlockSpec((1, E), lambda i: (0, 0)),
        ],
        out_specs=pl.BlockSpec((blk, E), lambda i: (i, 0)),
        out_shape=jax.ShapeDtypeStruct((V, E), jnp.float32),
    )(table, W, b.reshape(1, E))


def _make_gather(B, E, V):
    assert B % (_NW * _GROUP) == 0
    b_per_w = B // _NW
    n_groups = b_per_w // _GROUP
    assert n_groups % 2 == 0 and n_groups >= 8
    v_span = (V + _NS * 8 - 1) // (_NS * 8) * 8
    v_last = V - v_span * (_NS - 1)
    assert 0 < v_last <= v_span and v_last % 8 == 0
    mesh = plsc.VectorSubcoreMesh(
        core_axis_name="c", subcore_axis_name="s",
        num_cores=_NC, num_subcores=_NS,
    )

    @functools.partial(
        pl.kernel,
        mesh=mesh,
        out_type=jax.ShapeDtypeStruct((B, E), jnp.float32),
        scratch_types=[
            pltpu.VMEM((_GROUP,), jnp.int32),
            pltpu.VMEM((_GROUP,), jnp.int32),
            pltpu.VMEM((_GROUP, E), jnp.float32),
            pltpu.VMEM((_GROUP, E), jnp.float32),
            pltpu.VMEM_SHARED((V, E), jnp.float32),
            pltpu.SemaphoreType.DMA,
            pltpu.SemaphoreType.DMA,
            pltpu.SemaphoreType.DMA,
            pltpu.SemaphoreType.DMA,
            pltpu.SemaphoreType.DMA,
            pltpu.SemaphoreType.DMA,
        ],
    )
    def gather_kernel(idx_hbm, proj_hbm, out_hbm,
                      idx_a, idx_b, rows_a, rows_b, proj_sp,
                      si0, si1, sg0, sg1, ss0, ss1):
        idx = [idx_a, idx_b]
        rows = [rows_a, rows_b]
        sem_i = [si0, si1]
        sem_g = [sg0, sg1]
        sem_st = [ss0, ss1]
        sid = lax.axis_index("s")
        wid = sid * _NC + lax.axis_index("c")
        base = wid * b_per_w

        pltpu.async_copy(idx_hbm.at[pl.ds(base, _GROUP)], idx_a, si0)
        pltpu.async_copy(idx_hbm.at[pl.ds(base + _GROUP, _GROUP)], idx_b, si1)

        r0 = sid * v_span

        @pl.when(sid < _NS - 1)
        def _():
            pltpu.sync_copy(proj_hbm.at[pl.ds(r0, v_span)],
                            proj_sp.at[pl.ds(r0, v_span)])

        @pl.when(sid == _NS - 1)
        def _():
            pltpu.sync_copy(proj_hbm.at[pl.ds(v_span * (_NS - 1), v_last)],
                            proj_sp.at[pl.ds(v_span * (_NS - 1), v_last)])

        plsc.subcore_barrier()

        def fire_idx(g, p):
            pltpu.async_copy(
                idx_hbm.at[pl.ds(base + g * _GROUP, _GROUP)],
                idx[p], sem_i[p])

        def wait_idx(g, p):
            pltpu.make_async_copy(
                idx_hbm.at[pl.ds(base + g * _GROUP, _GROUP)],
                idx[p], sem_i[p]).wait()

        def fire_gathers(g, p):
            pltpu.async_copy(proj_sp.at[idx[p]], rows[p], sem_g[p])

        def wait_gathers(g, p):
            pltpu.make_async_copy(
                proj_sp.at[idx[p]], rows[p], sem_g[p]).wait()

        def fire_store(g, p):
            pltpu.async_copy(
                rows[p], out_hbm.at[pl.ds(base + g * _GROUP, _GROUP)],
                sem_st[p])

        def wait_store(g, p):
            pltpu.make_async_copy(
                rows[p], out_hbm.at[pl.ds(base + g * _GROUP, _GROUP)],
                sem_st[p]).wait()

        wait_idx(0, 0)
        fire_gathers(0, 0)
        wait_idx(1, 1)
        fire_gathers(1, 1)
        wait_gathers(0, 0)
        fire_idx(2, 0)
        fire_store(0, 0)

        def pair(t, carry):
            g = 2 * t
            wait_store(g - 2, 0)
            wait_idx(g, 0)
            fire_gathers(g, 0)
            wait_gathers(g - 1, 1)
            fire_idx(g + 1, 1)
            fire_store(g - 1, 1)
            wait_store(g - 1, 1)
            wait_idx(g + 1, 1)
            fire_gathers(g + 1, 1)
            wait_gathers(g, 0)
            fire_idx(g + 2, 0)
            fire_store(g, 0)
            return carry

        lax.fori_loop(1, n_groups // 2 - 1, pair, 0)

        g = n_groups - 2
        wait_store(g - 2, 0)
        wait_idx(g, 0)
        fire_gathers(g, 0)
        wait_gathers(g - 1, 1)
        fire_idx(g + 1, 1)
        fire_store(g - 1, 1)
        wait_store(g - 1, 1)
        wait_idx(g + 1, 1)
        fire_gathers(g + 1, 1)
        wait_gathers(g, 0)
        fire_store(g, 0)
        wait_gathers(g + 1, 1)
        fire_store(g + 1, 1)
        wait_store(g, 0)
        wait_store(g + 1, 1)

    return gather_kernel


def kernel(X, table, W, b):
    Bt, S = X.shape
    V = table.shape[0]
    proj = _project_table(table, W, b)
    idx = X.reshape(-1).astype(jnp.int32)
    out = _make_gather(idx.shape[0], _EMBED, V)(idx, proj)
    return out.reshape(Bt, S, _EMBED)


# speedup vs baseline: 1.0066x; 1.0066x over previous
import functools

import jax
import jax.numpy as jnp
from jax import lax
from jax.experimental import pallas as pl
from jax.experimental.pallas import tpu as pltpu
from jax.experimental.pallas import tpu_sc as plsc

_NC = 2
_NS = 16
_NW = _NC * _NS

_EMBED = 128
_GROUP = 160


def _proj_body(table_ref, w_ref, b_ref, out_ref):
    out_ref[...] = (
        jnp.dot(table_ref[...], w_ref[...], preferred_element_type=jnp.float32)
        + b_ref[...]
    )


def _project_table(table, W, b):
    V, _ = table.shape
    E = W.shape[1]
    return pl.pallas_call(
        _proj_body,
        out_shape=jax.ShapeDtypeStruct((V, E), jnp.float32),
    )(table, W, b.reshape(1, E))


def _make_gather(B, E, V):
    assert B % (_NW * _GROUP) == 0
    b_per_w = B // _NW
    n_groups = b_per_w // _GROUP
    assert n_groups % 2 == 0 and n_groups >= 8
    v_span = (V + _NS * 8 - 1) // (_NS * 8) * 8
    v_last = V - v_span * (_NS - 1)
    assert 0 < v_last <= v_span and v_last % 8 == 0
    mesh = plsc.VectorSubcoreMesh(
        core_axis_name="c", subcore_axis_name="s",
        num_cores=_NC, num_subcores=_NS,
    )

    @functools.partial(
        pl.kernel,
        mesh=mesh,
        out_type=jax.ShapeDtypeStruct((B, E), jnp.float32),
        scratch_types=[
            pltpu.VMEM((_GROUP,), jnp.int32),
            pltpu.VMEM((_GROUP,), jnp.int32),
            pltpu.VMEM((_GROUP, E), jnp.float32),
            pltpu.VMEM((_GROUP, E), jnp.float32),
            pltpu.VMEM_SHARED((V, E), jnp.float32),
            pltpu.SemaphoreType.DMA,
            pltpu.SemaphoreType.DMA,
            pltpu.SemaphoreType.DMA,
            pltpu.SemaphoreType.DMA,
            pltpu.SemaphoreType.DMA,
            pltpu.SemaphoreType.DMA,
        ],
    )
    def gather_kernel(idx_hbm, proj_hbm, out_hbm,
                      idx_a, idx_b, rows_a, rows_b, proj_sp,
                      si0, si1, sg0, sg1, ss0, ss1):
        idx = [idx_a, idx_b]
        rows = [rows_a, rows_b]
        sem_i = [si0, si1]
        sem_g = [sg0, sg1]
        sem_st = [ss0, ss1]
        sid = lax.axis_index("s")
        wid = sid * _NC + lax.axis_index("c")
        base = wid * b_per_w

        pltpu.async_copy(idx_hbm.at[pl.ds(base, _GROUP)], idx_a, si0)
        pltpu.async_copy(idx_hbm.at[pl.ds(base + _GROUP, _GROUP)], idx_b, si1)

        r0 = sid * v_span

        @pl.when(sid < _NS - 1)
        def _():
            pltpu.sync_copy(proj_hbm.at[pl.ds(r0, v_span)],
                            proj_sp.at[pl.ds(r0, v_span)])

        @pl.when(sid == _NS - 1)
        def _():
            pltpu.sync_copy(proj_hbm.at[pl.ds(v_span * (_NS - 1), v_last)],
                            proj_sp.at[pl.ds(v_span * (_NS - 1), v_last)])

        plsc.subcore_barrier()

        def fire_idx(g, p):
            pltpu.async_copy(
                idx_hbm.at[pl.ds(base + g * _GROUP, _GROUP)],
                idx[p], sem_i[p])

        def wait_idx(g, p):
            pltpu.make_async_copy(
                idx_hbm.at[pl.ds(base + g * _GROUP, _GROUP)],
                idx[p], sem_i[p]).wait()

        def fire_gathers(g, p):
            pltpu.async_copy(proj_sp.at[idx[p]], rows[p], sem_g[p])

        def wait_gathers(g, p):
            pltpu.make_async_copy(
                proj_sp.at[idx[p]], rows[p], sem_g[p]).wait()

        def fire_store(g, p):
            pltpu.async_copy(
                rows[p], out_hbm.at[pl.ds(base + g * _GROUP, _GROUP)],
                sem_st[p])

        def wait_store(g, p):
            pltpu.make_async_copy(
                rows[p], out_hbm.at[pl.ds(base + g * _GROUP, _GROUP)],
                sem_st[p]).wait()

        wait_idx(0, 0)
        fire_gathers(0, 0)
        wait_idx(1, 1)
        fire_gathers(1, 1)
        wait_gathers(0, 0)
        fire_idx(2, 0)
        fire_store(0, 0)

        def pair(t, carry):
            g = 2 * t
            wait_store(g - 2, 0)
            wait_idx(g, 0)
            fire_gathers(g, 0)
            wait_gathers(g - 1, 1)
            fire_idx(g + 1, 1)
            fire_store(g - 1, 1)
            wait_store(g - 1, 1)
            wait_idx(g + 1, 1)
            fire_gathers(g + 1, 1)
            wait_gathers(g, 0)
            fire_idx(g + 2, 0)
            fire_store(g, 0)
            return carry

        lax.fori_loop(1, n_groups // 2 - 1, pair, 0)

        g = n_groups - 2
        wait_store(g - 2, 0)
        wait_idx(g, 0)
        fire_gathers(g, 0)
        wait_gathers(g - 1, 1)
        fire_idx(g + 1, 1)
        fire_store(g - 1, 1)
        wait_store(g - 1, 1)
        wait_idx(g + 1, 1)
        fire_gathers(g + 1, 1)
        wait_gathers(g, 0)
        fire_store(g, 0)
        wait_gathers(g + 1, 1)
        fire_store(g + 1, 1)
        wait_store(g, 0)
        wait_store(g + 1, 1)

    return gather_kernel


def kernel(X, table, W, b):
    Bt, S = X.shape
    V = table.shape[0]
    proj = _project_table(table, W, b)
    idx = X.reshape(-1).astype(jnp.int32)
    out = _make_gather(idx.shape[0], _EMBED, V)(idx, proj)
    return out.reshape(Bt, S, _EMBED)
